# baseline (speedup 1.0000x reference)
import jax
import jax.numpy as jnp
from jax.experimental import pallas as pl

D_MODEL_ = 1024
NUM_EXPERTS_ = 8
D_FF_ = 2048
N_FF_CHUNKS_ = 4
FF_CHUNK_ = D_FF_ // N_FF_CHUNKS_


def _moe_kernel(x_ref, w1_ref, w2_ref, rw_ref, out_ref):
    e = pl.program_id(0)
    f = pl.program_id(1)
    x = x_ref[...]

    logits = jax.lax.dot_general(
        x, rw_ref[...], (((1,), (1,)), ((), ())),
        preferred_element_type=jnp.float32)
    col = jax.lax.broadcasted_iota(jnp.int32, logits.shape, 1)
    m1 = jnp.max(logits, axis=1, keepdims=True)
    i1 = jnp.min(jnp.where(logits == m1, col, NUM_EXPERTS_), axis=1,
                 keepdims=True)
    masked = jnp.where(col == i1, -jnp.inf, logits)
    m2 = jnp.max(masked, axis=1, keepdims=True)
    i2 = jnp.min(jnp.where(masked == m2, col, NUM_EXPERTS_), axis=1,
                 keepdims=True)
    p1 = 1.0 / (1.0 + jnp.exp(m2 - m1))
    p2 = 1.0 - p1
    c_e = jnp.sum(jnp.where(col == i1, p1, 0.0) * (col == e)
                  + jnp.where(col == i2, p2, 0.0) * (col == e), axis=1)

    w1_ef = w1_ref[0]
    w2_ef = w2_ref[0]
    h = jax.lax.dot_general(x, w1_ef, (((1,), (1,)), ((), ())),
                            preferred_element_type=jnp.float32)
    h = 0.5 * h * (1.0 + jax.lax.erf(h * (2.0 ** -0.5)))
    y = jax.lax.dot_general(h, w2_ef, (((1,), (1,)), ((), ())),
                            preferred_element_type=jnp.float32)

    contrib = c_e[:, None] * y

    @pl.when((e == 0) & (f == 0))
    def _():
        out_ref[...] = contrib

    @pl.when((e > 0) | (f > 0))
    def _():
        out_ref[...] += contrib


@jax.jit
def _moe(x_flat, w1, w2, router_w):
    n = x_flat.shape[0]
    return pl.pallas_call(
        _moe_kernel,
        grid=(NUM_EXPERTS_, N_FF_CHUNKS_),
        in_specs=[
            pl.BlockSpec((n, D_MODEL_), lambda e, f: (0, 0)),
            pl.BlockSpec((1, FF_CHUNK_, D_MODEL_), lambda e, f: (e, f, 0)),
            pl.BlockSpec((1, D_MODEL_, FF_CHUNK_), lambda e, f: (e, 0, f)),
            pl.BlockSpec((NUM_EXPERTS_, D_MODEL_), lambda e, f: (0, 0)),
        ],
        out_specs=pl.BlockSpec((n, D_MODEL_), lambda e, f: (0, 0)),
        out_shape=jax.ShapeDtypeStruct((n, D_MODEL_), jnp.float32),
    )(x_flat, w1, w2, router_w)


def kernel(x, w1, w2, router_w):
    B, T, D = x.shape
    out = _moe(x.reshape(B * T, D), w1, w2, router_w)
    return out.reshape(B, T, D)


# speedup vs baseline: 1.1468x; 1.1468x over previous
import jax
import jax.numpy as jnp
from jax.experimental import pallas as pl

D_MODEL_ = 1024
NUM_EXPERTS_ = 8
D_FF_ = 2048
N_FF_CHUNKS_ = 1
FF_CHUNK_ = D_FF_ // N_FF_CHUNKS_


def _moe_kernel(x_ref, w1_ref, w2_ref, rw_ref, out_ref):
    e = pl.program_id(0)
    f = pl.program_id(1)
    x = x_ref[...]

    logits = jax.lax.dot_general(
        x, rw_ref[...], (((1,), (1,)), ((), ())),
        preferred_element_type=jnp.float32)
    col = jax.lax.broadcasted_iota(jnp.int32, logits.shape, 1)
    m1 = jnp.max(logits, axis=1, keepdims=True)
    i1 = jnp.min(jnp.where(logits == m1, col, NUM_EXPERTS_), axis=1,
                 keepdims=True)
    masked = jnp.where(col == i1, -jnp.inf, logits)
    m2 = jnp.max(masked, axis=1, keepdims=True)
    i2 = jnp.min(jnp.where(masked == m2, col, NUM_EXPERTS_), axis=1,
                 keepdims=True)
    p1 = 1.0 / (1.0 + jnp.exp(m2 - m1))
    p2 = 1.0 - p1
    c_e = jnp.sum(jnp.where(col == i1, p1, 0.0) * (col == e)
                  + jnp.where(col == i2, p2, 0.0) * (col == e), axis=1)

    w1_ef = w1_ref[0]
    w2_ef = w2_ref[0]
    h = jax.lax.dot_general(x, w1_ef, (((1,), (1,)), ((), ())),
                            preferred_element_type=jnp.float32)
    h = 0.5 * h * (1.0 + jax.lax.erf(h * (2.0 ** -0.5)))
    y = jax.lax.dot_general(h, w2_ef, (((1,), (1,)), ((), ())),
                            preferred_element_type=jnp.float32)

    contrib = c_e[:, None] * y

    @pl.when((e == 0) & (f == 0))
    def _():
        out_ref[...] = contrib

    @pl.when((e > 0) | (f > 0))
    def _():
        out_ref[...] += contrib


@jax.jit
def _moe(x_flat, w1, w2, router_w):
    n = x_flat.shape[0]
    return pl.pallas_call(
        _moe_kernel,
        grid=(NUM_EXPERTS_, N_FF_CHUNKS_),
        in_specs=[
            pl.BlockSpec((n, D_MODEL_), lambda e, f: (0, 0)),
            pl.BlockSpec((1, FF_CHUNK_, D_MODEL_), lambda e, f: (e, f, 0)),
            pl.BlockSpec((1, D_MODEL_, FF_CHUNK_), lambda e, f: (e, 0, f)),
            pl.BlockSpec((NUM_EXPERTS_, D_MODEL_), lambda e, f: (0, 0)),
        ],
        out_specs=pl.BlockSpec((n, D_MODEL_), lambda e, f: (0, 0)),
        out_shape=jax.ShapeDtypeStruct((n, D_MODEL_), jnp.float32),
    )(x_flat, w1, w2, router_w)


def kernel(x, w1, w2, router_w):
    B, T, D = x.shape
    out = _moe(x.reshape(B * T, D), w1, w2, router_w)
    return out.reshape(B, T, D)
